# baseline (speedup 1.0000x reference)
import functools

import jax
import jax.numpy as jnp
from jax import lax
from jax.experimental import pallas as pl
from jax.experimental.pallas import tpu as pltpu
from jax.experimental.pallas import tpu_sc as plsc

VOCAB = 100000
D_MODEL = 2048
SEQ = 8192
B_HASH = 16

_L = 16
_NC = 2
_NS = 16
_NW = _NC * _NS
_ROWS_PER_W = SEQ // _NW
_CHUNK = 16
_NCHUNK = _ROWS_PER_W // _CHUNK


def _sc_gather_sum(table, ids):
  mesh = plsc.VectorSubcoreMesh(core_axis_name="c", subcore_axis_name="s")

  @functools.partial(
      pl.kernel,
      mesh=mesh,
      out_type=jax.ShapeDtypeStruct((_NW, D_MODEL), jnp.float32),
      scratch_types=[
          pltpu.VMEM((_ROWS_PER_W,), jnp.int32),
          pltpu.VMEM((_CHUNK, D_MODEL), jnp.float32),
          pltpu.VMEM((_CHUNK, D_MODEL), jnp.float32),
          pltpu.VMEM((D_MODEL,), jnp.float32),
          pltpu.SemaphoreType.DMA,
          pltpu.SemaphoreType.DMA,
      ],
  )
  def k(table_hbm, ids_hbm, out_hbm, idx_v, buf0, buf1, acc_v, sem0, sem1):
    wid = lax.axis_index("s") * _NC + lax.axis_index("c")
    base = wid * _ROWS_PER_W
    pltpu.sync_copy(ids_hbm.at[pl.ds(base, _ROWS_PER_W)], idx_v)

    bufs = (buf0, buf1)
    sems = (sem0, sem1)
    handles = [None, None]
    handles[0] = pltpu.async_copy(
        table_hbm.at[idx_v.at[pl.ds(0, _CHUNK)]], buf0, sem0)

    def zero_body(g, _):
      acc_v[pl.ds(g * _L, _L)] = jnp.zeros((_L,), jnp.float32)
      return 0
    lax.fori_loop(0, D_MODEL // _L, zero_body, 0)

    for c in range(_NCHUNK):
      if c + 1 < _NCHUNK:
        handles[(c + 1) % 2] = pltpu.async_copy(
            table_hbm.at[idx_v.at[pl.ds((c + 1) * _CHUNK, _CHUNK)]],
            bufs[(c + 1) % 2], sems[(c + 1) % 2])
      handles[c % 2].wait()
      buf = bufs[c % 2]

      def g_body(g, _):
        def r_body(r, v):
          return v + buf[r, pl.ds(g * _L, _L)]
        acc_v[pl.ds(g * _L, _L)] = lax.fori_loop(
            0, _CHUNK, r_body, acc_v[pl.ds(g * _L, _L)])
        return 0
      lax.fori_loop(0, D_MODEL // _L, g_body, 0)

    pltpu.sync_copy(acc_v, out_hbm.at[wid])

  return k(table, ids)


def _threefry2x32(k1, k2, x0, x1):
  ks2 = k1 ^ k2 ^ jnp.int32(0x1BD11BDA)
  ks = (k1, k2, ks2)

  def rot(x, d):
    return lax.shift_left(x, jnp.int32(d)) | lax.shift_right_logical(
        x, jnp.int32(32 - d))

  rots = ((13, 15, 26, 6), (17, 29, 16, 24))
  x0 = x0 + ks[0]
  x1 = x1 + ks[1]
  for i in range(5):
    for d in rots[i % 2]:
      x0 = x0 + x1
      x1 = rot(x1, d) ^ x0
    x0 = x0 + ks[(i + 1) % 3]
    x1 = x1 + ks[(i + 2) % 3] + jnp.int32(i + 1)
  return x0, x1


def _tc_tail(partials_ref, r_ref, logits_ref, out_ref):
  mean = jnp.sum(partials_ref[...], axis=0, keepdims=True) * jnp.float32(
      1.0 / SEQ)
  proj = jnp.sum(r_ref[...] * mean, axis=1, keepdims=True)
  bits = (proj > 0).astype(jnp.int32)
  sh = jnp.int32(B_HASH - 1) - lax.broadcasted_iota(jnp.int32, (B_HASH, 1), 0)
  seed = jnp.sum(lax.shift_left(bits, sh))

  nk1, nk2 = _threefry2x32(jnp.int32(0), jnp.int32(0), jnp.int32(0), seed)

  cnt = lax.broadcasted_iota(jnp.int32, (1, VOCAB), 1)
  o0, o1 = _threefry2x32(nk1, nk2, jnp.zeros((1, VOCAB), jnp.int32), cnt)
  rbits = o0 ^ o1
  fbits = lax.shift_right_logical(rbits, 9) | jnp.int32(0x3F800000)
  x = lax.bitcast_convert_type(fbits, jnp.float32) - jnp.float32(1.0)

  l = logits_ref[...]
  m = jnp.max(l)
  e = jnp.exp(l - m)
  s = jnp.sum(e)
  score = -jnp.log(e / s) / x
  mn = jnp.min(score)
  idx = lax.broadcasted_iota(jnp.int32, (1, VOCAB), 1)
  tok = jnp.min(jnp.where(score == mn, idx, jnp.int32(2**30)))
  out_ref[...] = jnp.where(idx == tok, jnp.float32(100000.0),
                           jnp.float32(-100000.0))


def kernel(input_ids, logits, embed_table, r_vectors):
  ids = input_ids.reshape(SEQ).astype(jnp.int32)
  partials = _sc_gather_sum(embed_table, ids)
  out = pl.pallas_call(
      _tc_tail,
      out_shape=jax.ShapeDtypeStruct((1, VOCAB), jnp.float32),
  )(partials, r_vectors, logits)
  return out


# speedup vs baseline: 2.9537x; 2.9537x over previous
import functools

import jax
import jax.numpy as jnp
from jax import lax
from jax.experimental import pallas as pl
from jax.experimental.pallas import tpu as pltpu
from jax.experimental.pallas import tpu_sc as plsc

VOCAB = 100000
D_MODEL = 2048
SEQ = 8192
B_HASH = 16

_L = 16
_NC = 2
_NS = 16
_NW = _NC * _NS
_ROWS_PER_W = SEQ // _NW
_CHUNK = 16
_NCHUNK = _ROWS_PER_W // _CHUNK


def _sc_gather_sum(table, ids):
  mesh = plsc.VectorSubcoreMesh(core_axis_name="c", subcore_axis_name="s")

  @functools.partial(
      pl.kernel,
      mesh=mesh,
      out_type=jax.ShapeDtypeStruct((_NW, D_MODEL), jnp.float32),
      scratch_types=[
          pltpu.VMEM((_ROWS_PER_W,), jnp.int32),
          pltpu.VMEM((_CHUNK, D_MODEL), jnp.float32),
          pltpu.VMEM((_CHUNK, D_MODEL), jnp.float32),
          pltpu.VMEM((D_MODEL,), jnp.float32),
          pltpu.SemaphoreType.DMA,
          pltpu.SemaphoreType.DMA,
      ],
  )
  def k(table_hbm, ids_hbm, out_hbm, idx_v, buf0, buf1, acc_v, sem0, sem1):
    wid = lax.axis_index("s") * _NC + lax.axis_index("c")
    base = wid * _ROWS_PER_W
    pltpu.sync_copy(ids_hbm.at[pl.ds(base, _ROWS_PER_W)], idx_v)

    bufs = (buf0, buf1)
    sems = (sem0, sem1)
    handles = [None, None]
    handles[0] = pltpu.async_copy(
        table_hbm.at[idx_v.at[pl.ds(0, _CHUNK)]], buf0, sem0)

    for c in range(_NCHUNK):
      if c + 1 < _NCHUNK:
        handles[(c + 1) % 2] = pltpu.async_copy(
            table_hbm.at[idx_v.at[pl.ds((c + 1) * _CHUNK, _CHUNK)]],
            bufs[(c + 1) % 2], sems[(c + 1) % 2])
      handles[c % 2].wait()
      buf = bufs[c % 2]
      first = (c == 0)

      @plsc.parallel_loop(0, D_MODEL // _L, unroll=4)
      def _(g, _buf=buf, _first=first):
        sl = pl.ds(g * _L, _L)
        v = _buf[0, sl] if _first else acc_v[sl] + _buf[0, sl]
        for r in range(1, _CHUNK):
          v = v + _buf[r, sl]
        acc_v[sl] = v

    pltpu.sync_copy(acc_v, out_hbm.at[wid])

  return k(table, ids)


def _threefry2x32(k1, k2, x0, x1):
  ks2 = k1 ^ k2 ^ jnp.int32(0x1BD11BDA)
  ks = (k1, k2, ks2)

  def rot(x, d):
    return lax.shift_left(x, jnp.int32(d)) | lax.shift_right_logical(
        x, jnp.int32(32 - d))

  rots = ((13, 15, 26, 6), (17, 29, 16, 24))
  x0 = x0 + ks[0]
  x1 = x1 + ks[1]
  for i in range(5):
    for d in rots[i % 2]:
      x0 = x0 + x1
      x1 = rot(x1, d) ^ x0
    x0 = x0 + ks[(i + 1) % 3]
    x1 = x1 + ks[(i + 2) % 3] + jnp.int32(i + 1)
  return x0, x1


_VR = 8
_VC = VOCAB // _VR


def _tc_tail(partials_ref, r_ref, logits_ref, out_ref):
  mean = jnp.sum(partials_ref[...], axis=0, keepdims=True) * jnp.float32(
      1.0 / SEQ)
  proj = jnp.sum(r_ref[...] * mean, axis=1, keepdims=True)
  bits = (proj > 0).astype(jnp.int32)
  sh = jnp.int32(B_HASH - 1) - lax.broadcasted_iota(jnp.int32, (B_HASH, 1), 0)
  seed = jnp.sum(lax.shift_left(bits, sh))

  nk1, nk2 = _threefry2x32(jnp.int32(0), jnp.int32(0), jnp.int32(0), seed)

  idx = (lax.broadcasted_iota(jnp.int32, (_VR, _VC), 0) * jnp.int32(_VC)
         + lax.broadcasted_iota(jnp.int32, (_VR, _VC), 1))
  o0, o1 = _threefry2x32(nk1, nk2, jnp.zeros((_VR, _VC), jnp.int32), idx)
  rbits = o0 ^ o1
  fbits = lax.shift_right_logical(rbits, 9) | jnp.int32(0x3F800000)
  x = lax.bitcast_convert_type(fbits, jnp.float32) - jnp.float32(1.0)

  l = logits_ref[...]
  m = jnp.max(l)
  e = jnp.exp(l - m)
  s = jnp.sum(e)
  score = -jnp.log(e / s) / x
  mn = jnp.min(score)
  tok = jnp.min(jnp.where(score == mn, idx, jnp.int32(2**30)))
  out_ref[...] = jnp.where(idx == tok, jnp.float32(100000.0),
                           jnp.float32(-100000.0))


def kernel(input_ids, logits, embed_table, r_vectors):
  ids = input_ids.reshape(SEQ).astype(jnp.int32)
  partials = _sc_gather_sum(embed_table, ids)
  out2d = pl.pallas_call(
      _tc_tail,
      out_shape=jax.ShapeDtypeStruct((_VR, _VC), jnp.float32),
  )(partials, r_vectors, logits.reshape(_VR, _VC))
  return out2d.reshape(1, VOCAB)
